# baseline (speedup 1.0000x reference)
import dataclasses

import jax
import jax.numpy as jnp
from jax import lax
from jax.experimental import pallas as pl
from jax.experimental.pallas import tpu as pltpu
from jax.experimental.pallas import tpu_sc as plsc

B = 16384
T = 20
D = 64
L = 16
NC = 2
NS = 16
NW = NC * NS
B_PER_W = B // NW
NB = 32
NCHUNK = B_PER_W // NB
ROWS = NB * T
IDX_W = 128
N_IDX_ROWS = ROWS // IDX_W


def _sc_kernel(cw_hbm, tw_hbm, ein_hbm, eout_hbm, out_hbm,
               cen_idx_v, tgt_idx_v, cen_rows_v, tgt_rows_v, out_v, sem):
    wid = lax.axis_index("s") * NC + lax.axis_index("c")

    @pl.loop(0, NCHUNK)
    def _chunk(ci):
        base_b = wid * B_PER_W + ci * NB

        pltpu.sync_copy(cw_hbm.at[pl.ds(base_b, NB)], cen_idx_v)
        for k in range(N_IDX_ROWS):
            pltpu.sync_copy(tw_hbm.at[pl.ds(base_b * T + k * IDX_W, IDX_W)],
                            tgt_idx_v.at[k])

        cen_cp = pltpu.async_copy(ein_hbm.at[cen_idx_v], cen_rows_v, sem)
        for k in range(N_IDX_ROWS):
            pltpu.async_copy(
                eout_hbm.at[tgt_idx_v.at[k]],
                tgt_rows_v.at[pl.ds(k * IDX_W, IDX_W)], sem).wait()
        cen_cp.wait()

        lane = lax.broadcasted_iota(jnp.int32, (L,), 0)

        @pl.loop(0, NB // 4)
        def _quad(b4):
            bl = b4 * 4
            cen = [[cen_rows_v[bl + i, pl.ds(k * L, L)] for k in range(4)]
                   for i in range(4)]
            rbase = bl * T
            for g in range(5):
                w = jnp.zeros((L,), jnp.float32)
                for s in range(L):
                    j = g * L + s
                    i, t = j // T, j % T
                    r = rbase + i * T + t
                    acc = tgt_rows_v[r, pl.ds(0, L)] * cen[i][0]
                    acc += tgt_rows_v[r, pl.ds(L, L)] * cen[i][1]
                    acc += tgt_rows_v[r, pl.ds(2 * L, L)] * cen[i][2]
                    acc += tgt_rows_v[r, pl.ds(3 * L, L)] * cen[i][3]
                    w = jnp.where(lane == s, jnp.sum(acc), w)
                out_v[pl.ds(rbase + g * L, L)] = w

        pltpu.sync_copy(out_v, out_hbm.at[pl.ds(base_b * T, ROWS)])


def kernel(center_word, target_word, embedding_in, embedding_out):
    cw = center_word.reshape(B)
    tw = target_word.reshape(B * T)

    cp = pltpu.CompilerParams()
    for fld, val in (("needs_layout_passes", False),
                     ("use_tc_tiling_on_sc", False)):
        if fld in pltpu.CompilerParams.__dataclass_fields__:
            cp = dataclasses.replace(cp, **{fld: val})
    mesh = plsc.VectorSubcoreMesh(core_axis_name="c", subcore_axis_name="s")
    run = pl.kernel(
        _sc_kernel,
        out_type=jax.ShapeDtypeStruct((B * T,), jnp.float32),
        mesh=mesh,
        scratch_types=[
            pltpu.VMEM((NB,), jnp.int32),
            pltpu.VMEM((N_IDX_ROWS, IDX_W), jnp.int32),
            pltpu.VMEM((NB, D), jnp.float32),
            pltpu.VMEM((ROWS, D), jnp.float32),
            pltpu.VMEM((ROWS,), jnp.float32),
            pltpu.SemaphoreType.DMA,
        ],
        compiler_params=cp,
    )
    flat = run(cw, tw, embedding_in, embedding_out)
    return flat.reshape(B, T)
